# baseline (speedup 1.0000x reference)
import math

import jax
import jax.numpy as jnp
from jax.experimental import pallas as pl
from jax.experimental.pallas import tpu as pltpu

_K0 = math.sqrt(2.0 / math.pi)
_K1 = 0.044715 * _K0

_CAP = 3072
_OCAP = 1536
_SIZES = [512, 1024] + [3072] * 10 + [512]


def _gelu(x):
    x2 = x * x
    z = x * (_K1 * x2 + _K0)
    hx = 0.5 * x
    return hx + hx * jnp.tanh(z)


def _pieces(sz):
    out = []
    p0 = 0
    while p0 < sz:
        out.append((p0, min(_OCAP, sz - p0)))
        p0 += _OCAP
    return out


def _pipeline_body(x_hbm, o_hbm, buf, obuf, in_sem, out_sem):
    n = len(_SIZES)
    offs = [0]
    for sz in _SIZES:
        offs.append(offs[-1] + sz)

    def in_copy(i, slot):
        return pltpu.make_async_copy(
            x_hbm.at[pl.ds(offs[i], _SIZES[i]), :],
            buf.at[slot, pl.ds(0, _SIZES[i]), :],
            in_sem.at[slot])

    in_copy(0, 0).start()
    in_copy(1, 1).start()
    started = 2
    prev_out = None

    for i in range(n):
        slot = i % 2
        in_copy(i, slot).wait()
        for (p0, plen) in _pieces(_SIZES[i]):
            if prev_out is not None:
                prev_out.wait()
            obuf[pl.ds(0, plen), :] = _gelu(buf[slot, pl.ds(p0, plen), :])
            prev_out = pltpu.make_async_copy(
                obuf.at[pl.ds(0, plen), :],
                o_hbm.at[pl.ds(offs[i] + p0, plen), :],
                out_sem)
            prev_out.start()
        if started < n:
            in_copy(started, slot).start()
            started += 1

    prev_out.wait()


def kernel(x, log_tau, log_blend):
    B, T, D = x.shape
    rows = B * T
    x2 = x.reshape(rows, D)
    out = pl.pallas_call(
        _pipeline_body,
        in_specs=[pl.BlockSpec(memory_space=pltpu.MemorySpace.HBM)],
        out_specs=pl.BlockSpec(memory_space=pltpu.MemorySpace.HBM),
        out_shape=jax.ShapeDtypeStruct((rows, D), x.dtype),
        scratch_shapes=[
            pltpu.VMEM((2, _CAP, D), jnp.float32),
            pltpu.VMEM((_OCAP, D), jnp.float32),
            pltpu.SemaphoreType.DMA((2,)),
            pltpu.SemaphoreType.DMA,
        ],
        compiler_params=pltpu.CompilerParams(
            vmem_limit_bytes=100 * 1024 * 1024,
        ),
    )(x2)
    return out.reshape(B, T, D)


# speedup vs baseline: 1.2422x; 1.2422x over previous
import math

import jax
import jax.numpy as jnp
from jax.experimental import pallas as pl
from jax.experimental.pallas import tpu as pltpu

_K0 = math.sqrt(2.0 / math.pi)
_K1 = 0.044715 * _K0


def _gelu_block(x_ref, o_ref):
    x = x_ref[...]
    x2 = x * x
    z = x * (_K1 * x2 + _K0)
    hx = 0.5 * x
    o_ref[...] = hx + hx * jnp.tanh(z)


def kernel(x, log_tau, log_blend):
    B, T, D = x.shape
    rows = B * T
    x2 = x.reshape(rows, D)
    block = 2016
    out = pl.pallas_call(
        _gelu_block,
        grid=(pl.cdiv(rows, block),),
        in_specs=[pl.BlockSpec((block, D), lambda i: (i, 0))],
        out_specs=pl.BlockSpec((block, D), lambda i: (i, 0)),
        out_shape=jax.ShapeDtypeStruct((rows, D), x.dtype),
        compiler_params=pltpu.CompilerParams(
            vmem_limit_bytes=100 * 1024 * 1024,
            dimension_semantics=("parallel",),
        ),
    )(x2)
    return out.reshape(B, T, D)
